# Initial kernel scaffold; baseline (speedup 1.0000x reference)
import jax
import jax.numpy as jnp
from jax.experimental import pallas as pl


def kernel(x, eye):
    raise NotImplementedError("write your pallas kernel here")


# speedup vs baseline: 1.7066x; 1.7066x over previous
import functools

import jax
import jax.numpy as jnp
from jax import lax
from jax.experimental import pallas as pl
from jax.experimental.pallas import tpu as pltpu
from jax.experimental.pallas import tpu_sc as plsc

B = 4096 * 20
D = 1000
NC = 2
NS = 16
NW = NC * NS
ROWS_PER_W = B // NW
C = 128
NCHUNK = ROWS_PER_W // C
L = 16


@functools.partial(
    pl.kernel,
    mesh=plsc.VectorSubcoreMesh(core_axis_name="c", subcore_axis_name="s"),
    compiler_params=pltpu.CompilerParams(needs_layout_passes=False),
    out_type=jax.ShapeDtypeStruct((B * D,), jnp.float32),
    scratch_types=[
        pltpu.VMEM((ROWS_PER_W,), jnp.int32),
        pltpu.VMEM((C * D,), jnp.float32),
    ],
)
def _onehot_sc(x_hbm, z_hbm, out_hbm, idx_v, buf_v):
    cid = lax.axis_index("c")
    sid = lax.axis_index("s")
    wid = sid * NC + cid
    row0 = wid * ROWS_PER_W
    pltpu.sync_copy(x_hbm.at[pl.ds(row0, ROWS_PER_W)], idx_v)
    pltpu.sync_copy(z_hbm, buf_v)

    lanes = lax.iota(jnp.int32, L)
    ones = jnp.full((L,), 1.0, jnp.float32)
    zeros = jnp.zeros((L,), jnp.float32)

    def chunk_body(c, carry):
        for j in range(C // L):
            cols = idx_v[pl.ds(c * C + j * L, L)]
            pos = (j * L + lanes) * D + cols
            plsc.store_scatter(buf_v, [pos], ones)
        pltpu.sync_copy(buf_v, out_hbm.at[pl.ds((row0 + c * C) * D, C * D)])
        for j in range(C // L):
            cols = idx_v[pl.ds(c * C + j * L, L)]
            pos = (j * L + lanes) * D + cols
            plsc.store_scatter(buf_v, [pos], zeros)
        return carry

    lax.fori_loop(0, NCHUNK, chunk_body, 0)


def kernel(x, eye):
    del eye
    xf = x.reshape(-1)
    zeros = jnp.zeros((C * D,), jnp.float32)
    out = _onehot_sc(xf, zeros)
    return out.reshape(x.shape[0], x.shape[1], D)
